# baseline (speedup 1.0000x reference)
import jax
import jax.numpy as jnp
from jax import lax
from jax.experimental import pallas as pl
from jax.experimental.pallas import tpu as pltpu
from jax.experimental.pallas import tpu_sc as plsc

_F = 250
_NB = 20
_N = _F * _NB
_T = 640
_G = (_N + _T - 1) // _T
_NW = 32
_TROWS = 160
_TPAD = _NW * _TROWS


def _sc_band_body(a_hbm, b_hbm, band_hbm, atab, btab, vbuf, sema, semb):
    w = lax.axis_index("s") * 2 + lax.axis_index("c")
    iota = lax.iota(jnp.int32, 16)
    c0 = jnp.full((16,), 0, jnp.int32)
    c1 = jnp.full((16,), 1, jnp.int32)
    c2 = jnp.full((16,), 2, jnp.int32)
    c3 = jnp.full((16,), 3, jnp.int32)
    tbase = _TROWS * w

    cpa = pltpu.make_async_copy(a_hbm.at[pl.ds(tbase, _TROWS), :], atab, sema)
    cpb = pltpu.make_async_copy(b_hbm.at[pl.ds(tbase, _TROWS), :], btab, semb)
    cpa.start()
    cpb.start()
    cpa.wait()
    cpb.wait()

    nrows = jnp.clip(_N - tbase, 0, _TROWS)
    npairs = nrows * _NB // 32

    def chunk(cc):
        f = 16 * cc + iota
        row = f // _NB
        j = f % _NB
        blk = tbase // _NB + row // _NB
        bi = (row // _NB) * _NB + j

        ax1 = plsc.load_gather(atab, [row, c0])
        ay1 = plsc.load_gather(atab, [row, c1])
        ax2 = plsc.load_gather(atab, [row, c2])
        ay2 = plsc.load_gather(atab, [row, c3])
        bx1 = plsc.load_gather(btab, [bi, c0])
        by1 = plsc.load_gather(btab, [bi, c1])
        bx2 = plsc.load_gather(btab, [bi, c2])
        by2 = plsc.load_gather(btab, [bi, c3])

        inter_x1 = jnp.maximum(ax1, bx1)
        inter_x2 = jnp.minimum(ax2, bx2)
        inter_y1 = jnp.maximum(ay1, by1)
        inter_y2 = jnp.minimum(ay2, by2)
        inter_area = (
            jnp.maximum(inter_x2 - inter_x1, 0.0)
            * jnp.maximum(inter_y2 - inter_y1, 0.0)
        )
        boxa_area = (ax2 - ax1 + 1.0) * (ay2 - ay1 + 1.0)
        boxb_area = (bx2 - bx1 + 1.0) * (bx2 - by1 + 1.0)
        iou = inter_area / (boxa_area + boxb_area - inter_area)

        val = jnp.where(blk != 248, iou, 0.0)
        plsc.store_scatter(vbuf, [row, j], val)

    def pair(q, carry):
        chunk(2 * q)
        chunk(2 * q + 1)
        return carry

    lax.fori_loop(0, npairs, pair, 0)
    pltpu.sync_copy(vbuf, band_hbm.at[pl.ds(tbase, _TROWS), :])


def _tc_strip_kernel(band_ref, o_ref):
    s = pl.program_id(0)

    band = band_ref[...]
    pj = jax.lax.broadcasted_iota(jnp.int32, (_NB, _T), 0)
    pc = jax.lax.broadcasted_iota(jnp.int32, (_NB, _T), 1) % _NB
    sel = jnp.where(pj == pc, 1.0, 0.0)
    tile = jax.lax.dot_general(
        band, sel, (((1,), (0,)), ((), ())),
        preferred_element_type=jnp.float32,
    )

    r = jax.lax.broadcasted_iota(jnp.int32, (_T, _T), 0) // _NB
    c = jax.lax.broadcasted_iota(jnp.int32, (_T, _T), 1) // _NB
    tile = jnp.where(r == c, tile, 0.0)

    o_ref[...] = jnp.zeros_like(o_ref)

    @pl.when(s < _G - 1)
    def _full():
        o_ref[:, pl.ds(s * _T, _T)] = tile

    @pl.when(s == _G - 1)
    def _last():
        o_ref[:, pl.ds(s * _T, _N - (_G - 1) * _T)] = tile[:, : _N - (_G - 1) * _T]


def kernel(rois):
    a_tbl = jnp.roll(rois, -1, axis=0).reshape(_N, 4)
    b_tbl = jnp.roll(rois, -2, axis=0).reshape(_N, 4)
    a_tbl = jnp.pad(a_tbl, ((0, _TPAD - _N), (0, 0)))
    b_tbl = jnp.pad(b_tbl, ((0, _TPAD - _N), (0, 0)))

    mesh = plsc.VectorSubcoreMesh(core_axis_name="c", subcore_axis_name="s")
    sc_band = pl.kernel(
        _sc_band_body,
        out_type=jax.ShapeDtypeStruct((_TPAD, _NB), jnp.float32),
        mesh=mesh,
        scratch_types=[
            pltpu.VMEM((_TROWS, 4), jnp.float32),
            pltpu.VMEM((_TROWS, 4), jnp.float32),
            pltpu.VMEM((_TROWS, _NB), jnp.float32),
            pltpu.SemaphoreType.DMA,
            pltpu.SemaphoreType.DMA,
        ],
        compiler_params=pltpu.CompilerParams(
            use_tc_tiling_on_sc=False, needs_layout_passes=False
        ),
    )
    band = sc_band(a_tbl, b_tbl)

    out = pl.pallas_call(
        _tc_strip_kernel,
        grid=(_G,),
        in_specs=[
            pl.BlockSpec((_T, _NB), lambda s: (s, 0)),
        ],
        out_specs=pl.BlockSpec((_T, _N), lambda s: (s, 0)),
        out_shape=jax.ShapeDtypeStruct((_N, _N), jnp.float32),
    )(band)
    return out.reshape(1, _N, _N)
